# baseline (speedup 1.0000x reference)
import functools

import jax
import jax.numpy as jnp
from jax import lax
from jax.experimental import pallas as pl
from jax.experimental.pallas import tpu as pltpu
from jax.experimental.pallas import tpu_sc as plsc

N = 10000
E = 320000
D = 128
H = 128
Z = 64
TAU = 0.8
NUM_CLUSTER = 20

NC = 2
NS = 16
NW = NC * NS
NP = 10240
RPT = NP // NS


def _seg_sum_multi_kernel(k, ch, g):
    per_w = E // NW
    n_ch = per_w // ch
    n_g = ch // g
    zr = min(ch - ch % 8, RPT)
    assert n_ch % 2 == 0
    mesh = plsc.VectorSubcoreMesh(core_axis_name="c", subcore_axis_name="s",
                                  num_cores=NC, num_subcores=NS)

    @functools.partial(
        pl.kernel,
        out_type=jax.ShapeDtypeStruct((k, NC, NP, 64), jnp.float32),
        mesh=mesh,
        compiler_params=pltpu.CompilerParams(use_tc_tiling_on_sc=False),
        scratch_types=[
            pltpu.VMEM((2, n_g, g), jnp.int32),
            pltpu.VMEM((2, n_g, g), jnp.int32),
            pltpu.VMEM((ch, 64), jnp.float32),
            pltpu.VMEM((ch, 64), jnp.float32),
            pltpu.VMEM_SHARED((NP, 64), jnp.float32),
            pltpu.SemaphoreType.DMA,
            pltpu.SemaphoreType.DMA,
        ],
    )
    def body(*refs):
        x_stk = refs[0]
        ers = refs[1:1 + k]
        zero_hbm = refs[1 + k]
        out_hbm = refs[2 + k]
        dst_v, src_v, rows_a, rows_b, acc_sh, sem_g, sem_s = refs[3 + k:]
        cid = lax.axis_index("c")
        sid = lax.axis_index("s")
        wid = sid * NC + cid

        for p in range(k):
            x_hbm, er_hbm = x_stk.at[p], ers[p]
            pltpu.sync_copy(zero_hbm, rows_a.at[pl.ds(0, zr)])
            for o in range(0, RPT, zr):
                m = min(zr, RPT - o)
                pltpu.sync_copy(rows_a.at[pl.ds(0, m)],
                                acc_sh.at[pl.ds(sid * RPT + o, m)])
            plsc.subcore_barrier()

            def pair(t, carry, x_hbm=x_hbm, er_hbm=er_hbm):
                a = 2 * t
                b = a + 1
                pltpu.sync_copy(er_hbm.at[0, wid, a], dst_v.at[0])
                pltpu.sync_copy(er_hbm.at[1, wid, a], src_v.at[0])
                ga = [
                    pltpu.async_copy(x_hbm.at[src_v.at[0, i]],
                                     rows_a.at[pl.ds(i * g, g)], sem_g)
                    for i in range(n_g)
                ]
                pltpu.sync_copy(er_hbm.at[0, wid, b], dst_v.at[1])
                pltpu.sync_copy(er_hbm.at[1, wid, b], src_v.at[1])
                for de in ga:
                    de.wait()
                sa = [
                    pltpu.async_copy(rows_a.at[pl.ds(i * g, g)],
                                     acc_sh.at[dst_v.at[0, i]], sem_s,
                                     add=True)
                    for i in range(n_g)
                ]
                gb = [
                    pltpu.async_copy(x_hbm.at[src_v.at[1, i]],
                                     rows_b.at[pl.ds(i * g, g)], sem_g)
                    for i in range(n_g)
                ]
                for de in gb:
                    de.wait()
                sb = [
                    pltpu.async_copy(rows_b.at[pl.ds(i * g, g)],
                                     acc_sh.at[dst_v.at[1, i]], sem_s,
                                     add=True)
                    for i in range(n_g)
                ]
                for de in sa + sb:
                    de.wait()
                return carry

            lax.fori_loop(0, n_ch // 2, pair, 0)
            plsc.subcore_barrier()

            for o in range(0, RPT, zr):
                m = min(zr, RPT - o)
                pltpu.sync_copy(acc_sh.at[pl.ds(sid * RPT + o, m)],
                                rows_a.at[pl.ds(0, m)])
                pltpu.sync_copy(rows_a.at[pl.ds(0, m)],
                                out_hbm.at[p, cid, pl.ds(sid * RPT + o, m)])
            if p + 1 < k:
                plsc.subcore_barrier()

    return body


_SS_CH = 500
_SS_G = 100


@functools.lru_cache(maxsize=None)
def _get_seg_sum_multi(k):
    return _seg_sum_multi_kernel(k, _SS_CH, _SS_G)


def _seg_sum_batch(x_stk, ers):
    zeros = jnp.zeros((min(_SS_CH - _SS_CH % 8, RPT), 64), jnp.float32)
    return _get_seg_sum_multi(len(ers))(x_stk, *ers, zeros)


_CNT_CH = 2000
_CNT_NCH = (E // NW) // _CNT_CH


def _make_count_kernel(n_lists):
    mesh = plsc.VectorSubcoreMesh(core_axis_name="c", subcore_axis_name="s",
                                  num_cores=NC, num_subcores=NS)

    @functools.partial(
        pl.kernel,
        out_type=jax.ShapeDtypeStruct((n_lists, NW, 1, NP), jnp.float32),
        mesh=mesh,
        compiler_params=pltpu.CompilerParams(use_tc_tiling_on_sc=False,
                                             needs_layout_passes=False),
        scratch_types=[
            pltpu.VMEM((_CNT_CH,), jnp.int32),
            pltpu.VMEM((NP,), jnp.float32),
        ],
    )
    def body(*refs):
        ers = refs[:n_lists]
        out_hbm = refs[n_lists]
        didx, cnt_v = refs[n_lists + 1:]
        cid = lax.axis_index("c")
        sid = lax.axis_index("s")
        wid = sid * NC + cid
        ones = jnp.ones((16,), jnp.float32)
        zeros = jnp.zeros((16,), jnp.float32)

        for l in range(n_lists):
            def zero(v, carry):
                cnt_v[pl.ds(v * 16, 16)] = zeros
                return carry
            lax.fori_loop(0, NP // 16, zero, 0)

            def chunk(c, carry, er=ers[l]):
                pltpu.sync_copy(er.at[wid, c, 0], didx)

                def group(v, carry2):
                    idx = didx[pl.ds(v * 16, 16)]
                    plsc.addupdate_scatter(cnt_v, [idx], ones)
                    return carry2

                lax.fori_loop(0, _CNT_CH // 16, group, 0)
                return carry

            lax.fori_loop(0, _CNT_NCH, chunk, 0)
            pltpu.sync_copy(cnt_v, out_hbm.at[l, wid, 0])

    return body


_get_count7 = functools.lru_cache(maxsize=None)(lambda: _make_count_kernel(7))


_BB = 2048


def _elu(x):
    return jnp.where(x > 0, x, jnp.exp(x) - 1.0)


def _encode_body(f0, fm, f1, f2, W0, b0, W1, b1, W2, b2,
                 ht_ref, hm_ref, hn0_ref, hn1_ref):
    elu = _elu
    ht_ref[...] = elu(f0[...] @ W0[...] + b0[...])
    hm_ref[...] = elu(fm[...] @ W0[...] + b0[...])
    hn0_ref[...] = elu(f1[...] @ W1[...] + b1[...])
    hn1_ref[...] = elu(f2[...] @ W2[...] + b2[...])


def _encode(f0p, fmp, f1p, f2p, W0, b0, W1, b1, W2, b2):
    bs_x = pl.BlockSpec((_BB, D), lambda i: (i, 0))
    bs_w = pl.BlockSpec((D, H), lambda i: (0, 0))
    bs_b = pl.BlockSpec((1, H), lambda i: (0, 0))
    return pl.pallas_call(
        _encode_body,
        grid=(NP // _BB,),
        in_specs=[bs_x, bs_x, bs_x, bs_x, bs_w, bs_b, bs_w, bs_b, bs_w, bs_b],
        out_specs=[pl.BlockSpec((_BB, H), lambda i: (i, 0))] * 4,
        out_shape=[jax.ShapeDtypeStruct((NP, H), jnp.float32)] * 4,
    )(f0p, fmp, f1p, f2p, W0, b0.reshape(1, H), W1, b1.reshape(1, H),
      W2, b2.reshape(1, H))


def _stage2_body(o1, r0, r1, aW0, aW1, ht, hm, W1, b1, ps_ref):
    elu = _elu
    a0 = jnp.concatenate([o1[0, 0] + o1[0, 1], o1[1, 0] + o1[1, 1]],
                         axis=1) * r0[...]
    a1 = jnp.concatenate([o1[2, 0] + o1[2, 1], o1[3, 0] + o1[3, 1]],
                         axis=1) * r1[...]
    t0 = a0 @ aW0[...]
    t1 = a1 @ aW1[...]
    hs = [elu(ht[...] + t0), elu(hm[...] + t0), elu(ht[...] + t1),
          elu(hm[...] + t1), ht[...]]
    for v in range(5):
        ps_ref[v] = hs[v] @ W1[...] + b1[...]


def _stage2(out1, r0, r1, aW0, aW1, ht, hm, W1, b1):
    return pl.pallas_call(
        _stage2_body,
        grid=(NP // _BB,),
        in_specs=[
            pl.BlockSpec((4, NC, _BB, 64), lambda i: (0, 0, i, 0)),
            pl.BlockSpec((_BB, 1), lambda i: (i, 0)),
            pl.BlockSpec((_BB, 1), lambda i: (i, 0)),
            pl.BlockSpec((H, H), lambda i: (0, 0)),
            pl.BlockSpec((H, H), lambda i: (0, 0)),
            pl.BlockSpec((_BB, H), lambda i: (i, 0)),
            pl.BlockSpec((_BB, H), lambda i: (i, 0)),
            pl.BlockSpec((H, Z), lambda i: (0, 0)),
            pl.BlockSpec((1, Z), lambda i: (0, 0)),
        ],
        out_specs=pl.BlockSpec((5, _BB, Z), lambda i: (0, i, 0)),
        out_shape=jax.ShapeDtypeStruct((5, NP, Z), jnp.float32),
    )(out1, r0, r1, aW0, aW1, ht, hm, W1, b1.reshape(1, Z))


def _stage3_body(o2, ra, rb, rc, rd, re, W2, b2, qs_ref):
    rs = [ra, rb, rc, rd, re]
    for v in range(5):
        m = (o2[v, 0] + o2[v, 1]) * rs[v][...]
        qs_ref[v] = jax.nn.relu(m) @ W2[...] + b2[...]


def _stage3(out2, rcols, W2, b2):
    return pl.pallas_call(
        _stage3_body,
        grid=(NP // _BB,),
        in_specs=[
            pl.BlockSpec((5, NC, _BB, 64), lambda i: (0, 0, i, 0)),
        ] + [pl.BlockSpec((_BB, 1), lambda i: (i, 0))] * 5 + [
            pl.BlockSpec((Z, Z), lambda i: (0, 0)),
            pl.BlockSpec((1, Z), lambda i: (0, 0)),
        ],
        out_specs=pl.BlockSpec((5, _BB, Z), lambda i: (0, i, 0)),
        out_shape=jax.ShapeDtypeStruct((5, NP, Z), jnp.float32),
    )(out2, *rcols, W2, b2.reshape(1, Z))


def _stage4_body(o3, ra, rb, rc, rd, re, pW, pb, zn_ref, zc_ref):
    i = pl.program_id(0)
    rowid = i * _BB + lax.broadcasted_iota(jnp.int32, (_BB, 1), 0)
    valid = rowid < N
    rs = [ra, rb, rc, rd, re]
    for v in range(4):
        m = (o3[v, 0] + o3[v, 1]) * rs[v][...]
        nrm = jnp.sqrt(jnp.sum(m * m, axis=1, keepdims=True)) + 1e-8
        zn_ref[v] = jnp.where(valid, m / nrm, 0.0)
    m4 = (o3[4, 0] + o3[4, 1]) * re[...]
    zc = jnp.tanh(m4 @ pW[...] + pb[...])
    nrm = jnp.sqrt(jnp.sum(zc * zc, axis=1, keepdims=True)) + 1e-8
    zc_ref[...] = jnp.where(valid, zc / nrm, 0.0)


def _stage4(out3, rcols, pW, pb):
    return pl.pallas_call(
        _stage4_body,
        grid=(NP // _BB,),
        in_specs=[
            pl.BlockSpec((5, NC, _BB, 64), lambda i: (0, 0, i, 0)),
        ] + [pl.BlockSpec((_BB, 1), lambda i: (i, 0))] * 5 + [
            pl.BlockSpec((Z, Z), lambda i: (0, 0)),
            pl.BlockSpec((1, Z), lambda i: (0, 0)),
        ],
        out_specs=[
            pl.BlockSpec((4, _BB, Z), lambda i: (0, i, 0)),
            pl.BlockSpec((_BB, Z), lambda i: (i, 0)),
        ],
        out_shape=[
            jax.ShapeDtypeStruct((4, NP, Z), jnp.float32),
            jax.ShapeDtypeStruct((NP, Z), jnp.float32),
        ],
    )(out3, *rcols, pW, pb.reshape(1, Z))


def _stage5_body(zn, beta, pW, pb, zp_ref):
    i = pl.program_id(0)
    rowid = i * _BB + lax.broadcasted_iota(jnp.int32, (_BB, 1), 0)
    valid = rowid < N
    z = (zn[0] * beta[0:1, 0:1] + zn[1] * beta[1:2, 0:1] +
         zn[2] * beta[2:3, 0:1] + zn[3] * beta[3:4, 0:1])
    zp = jnp.tanh(z @ pW[...] + pb[...])
    nrm = jnp.sqrt(jnp.sum(zp * zp, axis=1, keepdims=True)) + 1e-8
    zp_ref[...] = jnp.where(valid, zp / nrm, 0.0)


def _stage5(zn, beta, pW, pb):
    return pl.pallas_call(
        _stage5_body,
        grid=(NP // _BB,),
        in_specs=[
            pl.BlockSpec((4, _BB, Z), lambda i: (0, i, 0)),
            pl.BlockSpec((4, 1), lambda i: (0, 0)),
            pl.BlockSpec((Z, Z), lambda i: (0, 0)),
            pl.BlockSpec((1, Z), lambda i: (0, 0)),
        ],
        out_specs=pl.BlockSpec((_BB, Z), lambda i: (i, 0)),
        out_shape=jax.ShapeDtypeStruct((NP, Z), jnp.float32),
    )(zn, beta.reshape(4, 1), pW, pb.reshape(1, Z))


_BM = 1024


def _contrast_body(z1_ref, z2_ref, row_ref, col_ref, diag_ref):
    i = pl.program_id(0)
    j = pl.program_id(1)
    s = lax.dot_general(z1_ref[...].astype(jnp.bfloat16),
                        z2_ref[...].astype(jnp.bfloat16),
                        (((1,), (1,)), ((), ())),
                        preferred_element_type=jnp.float32) * (1.0 / TAU)
    e = jnp.exp(s)
    rs = jnp.sum(e, axis=1)
    cs = jnp.sum(e, axis=0)

    @pl.when(j == 0)
    def _():
        row_ref[0, pl.ds(i * _BM, _BM)] = rs

    @pl.when(j != 0)
    def _():
        row_ref[0, pl.ds(i * _BM, _BM)] += rs

    @pl.when(i == 0)
    def _():
        col_ref[0, pl.ds(j * _BM, _BM)] = cs

    @pl.when(i != 0)
    def _():
        col_ref[0, pl.ds(j * _BM, _BM)] += cs

    @pl.when(i == j)
    def _():
        diag_ref[0, pl.ds(i * _BM, _BM)] = (
            jnp.sum(z1_ref[...] * z2_ref[...], axis=1) * (1.0 / TAU))


def _contrast(z1p, z2p):
    pad = NP - N
    grid = (NP // _BM, NP // _BM)
    row, col, diag = pl.pallas_call(
        _contrast_body,
        grid=grid,
        in_specs=[
            pl.BlockSpec((_BM, Z), lambda i, j: (i, 0)),
            pl.BlockSpec((_BM, Z), lambda i, j: (j, 0)),
        ],
        out_specs=[
            pl.BlockSpec((1, NP), lambda i, j: (0, 0)),
            pl.BlockSpec((1, NP), lambda i, j: (0, 0)),
            pl.BlockSpec((1, NP), lambda i, j: (0, 0)),
        ],
        out_shape=[
            jax.ShapeDtypeStruct((1, NP), jnp.float32),
            jax.ShapeDtypeStruct((1, NP), jnp.float32),
            jax.ShapeDtypeStruct((1, NP), jnp.float32),
        ],
    )(z1p, z2p)
    return row[0, :N] - pad, col[0, :N] - pad, diag[0, :N]


def _reshape_edges(edge, ch, g):
    per_w = E // NW
    return edge.reshape(2, NW, per_w // ch, ch // g, g)


def _l2norm(x):
    return x / (jnp.linalg.norm(x, axis=1, keepdims=True) + 1e-8)


def kernel(feats_0, feats_1, feats_2, mask_feat, fc_W0, fc_b0, fc_W1, fc_b1,
           fc_W2, fc_b2, agg_W0, agg_W1, gcn_W1, gcn_b1, gcn_W2, gcn_b2,
           proj_W, proj_b, att_W, att_b, att_a, nei_edge_0, nei_edge_1,
           mask_edge_0, mask_edge_1, norm_edge_0, norm_edge_1, adj_edge,
           num_cluster):
    edges = [nei_edge_0, nei_edge_1, mask_edge_0, mask_edge_1, norm_edge_0,
             norm_edge_1, adj_edge]
    cnt_ers = [e[0].reshape(NW, _CNT_NCH, 1, _CNT_CH) for e in edges]
    cparts = _get_count7()(*cnt_ers)
    cnt = cparts[:, :, 0, :].sum(axis=1)
    recip = 1.0 / jnp.maximum(cnt, 1.0)
    rcol = [recip[v][:, None] for v in range(7)]
    r_nei0, r_nei1, r_mask0, r_mask1, r_norm0, r_norm1, r_adj = rcol

    er_nei = [_reshape_edges(e, _SS_CH, _SS_G) for e in (nei_edge_0,
                                                         nei_edge_1)]
    er5 = [_reshape_edges(e, _SS_CH, _SS_G) for e in
           (norm_edge_0, mask_edge_0, norm_edge_1, mask_edge_1, adj_edge)]
    rs5 = [r_norm0, r_mask0, r_norm1, r_mask1, r_adj]

    pad = ((0, NP - N), (0, 0))
    h_tar, h_mask, hn0, hn1 = _encode(
        jnp.pad(feats_0, pad), jnp.pad(mask_feat, pad), jnp.pad(feats_1, pad),
        jnp.pad(feats_2, pad), fc_W0, fc_b0, fc_W1, fc_b1, fc_W2, fc_b2)

    x1 = jnp.stack([hn0[:, :64], hn0[:, 64:], hn1[:, :64], hn1[:, 64:]])
    out1 = _seg_sum_batch(x1, [er_nei[0], er_nei[0], er_nei[1], er_nei[1]])

    ps = _stage2(out1, r_nei0, r_nei1, agg_W0, agg_W1, h_tar, h_mask,
                 gcn_W1, gcn_b1)

    out2 = _seg_sum_batch(ps, er5)
    qs = _stage3(out2, rs5, gcn_W2, gcn_b2)

    out3 = _seg_sum_batch(qs, er5)
    zn, z_coarse = _stage4(out3, rs5, proj_W, proj_b)

    zs = zn[:, :N]
    w = (jnp.tanh(zs @ att_W + att_b) @ att_a).mean(axis=1)
    beta = jax.nn.softmax(w)
    z_pro = _stage5(zn, beta, proj_W, proj_b)

    row_se, col_se, diag = _contrast(z_coarse, z_pro)
    l1 = (jnp.log(row_se) - diag).mean()
    l2_ = (jnp.log(col_se) - diag).mean()
    loss_info = 0.5 * (l1 + l2_)

    zp = z_pro[:N]
    assign = jnp.arange(N) % NUM_CLUSTER
    protos = _l2norm(zp.reshape(N // NUM_CLUSTER, NUM_CLUSTER, Z).mean(0))
    logits = zp @ protos.T / TAU
    pos = jnp.take_along_axis(logits, assign[:, None], axis=1)[:, 0]
    loss_proto = (jax.nn.logsumexp(logits, axis=1) - pos).mean()
    return loss_info + loss_proto


# speedup vs baseline: 1.0809x; 1.0809x over previous
import functools

import jax
import jax.numpy as jnp
from jax import lax
from jax.experimental import pallas as pl
from jax.experimental.pallas import tpu as pltpu
from jax.experimental.pallas import tpu_sc as plsc

N = 10000
E = 320000
D = 128
H = 128
Z = 64
TAU = 0.8
NUM_CLUSTER = 20

NC = 2
NS = 16
NW = NC * NS
NP = 10240
RPT = NP // NS


def _seg_sum_multi_kernel(k, ch, g):
    per_w = E // NW
    n_ch = per_w // ch
    n_g = ch // g
    zr = min(ch - ch % 8, RPT)
    assert n_ch % 2 == 0
    mesh = plsc.VectorSubcoreMesh(core_axis_name="c", subcore_axis_name="s",
                                  num_cores=NC, num_subcores=NS)

    @functools.partial(
        pl.kernel,
        out_type=jax.ShapeDtypeStruct((k, NC, NP, 64), jnp.float32),
        mesh=mesh,
        compiler_params=pltpu.CompilerParams(use_tc_tiling_on_sc=False),
        scratch_types=[
            pltpu.VMEM((n_ch, n_g, g), jnp.int32),
            pltpu.VMEM((n_ch, n_g, g), jnp.int32),
            pltpu.VMEM((ch, 64), jnp.float32),
            pltpu.VMEM((ch, 64), jnp.float32),
            pltpu.VMEM_SHARED((NP, 64), jnp.float32),
            pltpu.SemaphoreType.DMA,
            pltpu.SemaphoreType.DMA,
        ],
    )
    def body(*refs):
        x_stk = refs[0]
        ers = refs[1:1 + k]
        zero_hbm = refs[1 + k]
        out_hbm = refs[2 + k]
        dst_v, src_v, rows_a, rows_b, acc_sh, sem_g, sem_s = refs[3 + k:]
        cid = lax.axis_index("c")
        sid = lax.axis_index("s")
        wid = sid * NC + cid

        for p in range(k):
            x_hbm, er_hbm = x_stk.at[p], ers[p]
            pltpu.sync_copy(er_hbm.at[0, wid], dst_v)
            pltpu.sync_copy(er_hbm.at[1, wid], src_v)
            pltpu.sync_copy(zero_hbm, rows_a.at[pl.ds(0, zr)])
            for o in range(0, RPT, zr):
                m = min(zr, RPT - o)
                pltpu.sync_copy(rows_a.at[pl.ds(0, m)],
                                acc_sh.at[pl.ds(sid * RPT + o, m)])
            plsc.subcore_barrier()

            def pair(t, carry, x_hbm=x_hbm):
                a = 2 * t
                b = a + 1
                ga = [
                    pltpu.async_copy(x_hbm.at[src_v.at[a, i]],
                                     rows_a.at[pl.ds(i * g, g)], sem_g)
                    for i in range(n_g)
                ]
                for de in ga:
                    de.wait()
                sa = [
                    pltpu.async_copy(rows_a.at[pl.ds(i * g, g)],
                                     acc_sh.at[dst_v.at[a, i]], sem_s,
                                     add=True)
                    for i in range(n_g)
                ]
                gb = [
                    pltpu.async_copy(x_hbm.at[src_v.at[b, i]],
                                     rows_b.at[pl.ds(i * g, g)], sem_g)
                    for i in range(n_g)
                ]
                for de in gb:
                    de.wait()
                sb = [
                    pltpu.async_copy(rows_b.at[pl.ds(i * g, g)],
                                     acc_sh.at[dst_v.at[b, i]], sem_s,
                                     add=True)
                    for i in range(n_g)
                ]
                for de in sa + sb:
                    de.wait()
                return carry

            lax.fori_loop(0, n_ch // 2, pair, 0)
            plsc.subcore_barrier()

            for o in range(0, RPT, zr):
                m = min(zr, RPT - o)
                pltpu.sync_copy(acc_sh.at[pl.ds(sid * RPT + o, m)],
                                rows_a.at[pl.ds(0, m)])
                pltpu.sync_copy(rows_a.at[pl.ds(0, m)],
                                out_hbm.at[p, cid, pl.ds(sid * RPT + o, m)])
            if p + 1 < k:
                plsc.subcore_barrier()

    return body


_SS_CH = 500
_SS_G = 100


@functools.lru_cache(maxsize=None)
def _get_seg_sum_multi(k):
    return _seg_sum_multi_kernel(k, _SS_CH, _SS_G)


def _seg_sum_batch(x_stk, ers):
    zeros = jnp.zeros((min(_SS_CH - _SS_CH % 8, RPT), 64), jnp.float32)
    return _get_seg_sum_multi(len(ers))(x_stk, *ers, zeros)


_CNT_CH = 2000
_CNT_NCH = (E // NW) // _CNT_CH


def _make_count_kernel(n_lists):
    mesh = plsc.VectorSubcoreMesh(core_axis_name="c", subcore_axis_name="s",
                                  num_cores=NC, num_subcores=NS)

    @functools.partial(
        pl.kernel,
        out_type=jax.ShapeDtypeStruct((n_lists, NW, 1, NP), jnp.float32),
        mesh=mesh,
        compiler_params=pltpu.CompilerParams(use_tc_tiling_on_sc=False,
                                             needs_layout_passes=False),
        scratch_types=[
            pltpu.VMEM((_CNT_CH,), jnp.int32),
            pltpu.VMEM((NP,), jnp.float32),
        ],
    )
    def body(*refs):
        ers = refs[:n_lists]
        out_hbm = refs[n_lists]
        didx, cnt_v = refs[n_lists + 1:]
        cid = lax.axis_index("c")
        sid = lax.axis_index("s")
        wid = sid * NC + cid
        ones = jnp.ones((16,), jnp.float32)
        zeros = jnp.zeros((16,), jnp.float32)

        for l in range(n_lists):
            def zero(v, carry):
                cnt_v[pl.ds(v * 16, 16)] = zeros
                return carry
            lax.fori_loop(0, NP // 16, zero, 0)

            def chunk(c, carry, er=ers[l]):
                pltpu.sync_copy(er.at[wid, c, 0], didx)

                def group(v, carry2):
                    idx = didx[pl.ds(v * 16, 16)]
                    plsc.addupdate_scatter(cnt_v, [idx], ones)
                    return carry2

                lax.fori_loop(0, _CNT_CH // 16, group, 0)
                return carry

            lax.fori_loop(0, _CNT_NCH, chunk, 0)
            pltpu.sync_copy(cnt_v, out_hbm.at[l, wid, 0])

    return body


_get_count7 = functools.lru_cache(maxsize=None)(lambda: _make_count_kernel(7))


_BB = 2048


def _elu(x):
    return jnp.where(x > 0, x, jnp.exp(x) - 1.0)


def _encode_body(f0, fm, f1, f2, W0, b0, W1, b1, W2, b2,
                 ht_ref, hm_ref, hn0_ref, hn1_ref):
    elu = _elu
    ht_ref[...] = elu(f0[...] @ W0[...] + b0[...])
    hm_ref[...] = elu(fm[...] @ W0[...] + b0[...])
    hn0_ref[...] = elu(f1[...] @ W1[...] + b1[...])
    hn1_ref[...] = elu(f2[...] @ W2[...] + b2[...])


def _encode(f0p, fmp, f1p, f2p, W0, b0, W1, b1, W2, b2):
    bs_x = pl.BlockSpec((_BB, D), lambda i: (i, 0))
    bs_w = pl.BlockSpec((D, H), lambda i: (0, 0))
    bs_b = pl.BlockSpec((1, H), lambda i: (0, 0))
    return pl.pallas_call(
        _encode_body,
        grid=(NP // _BB,),
        in_specs=[bs_x, bs_x, bs_x, bs_x, bs_w, bs_b, bs_w, bs_b, bs_w, bs_b],
        out_specs=[pl.BlockSpec((_BB, H), lambda i: (i, 0))] * 4,
        out_shape=[jax.ShapeDtypeStruct((NP, H), jnp.float32)] * 4,
    )(f0p, fmp, f1p, f2p, W0, b0.reshape(1, H), W1, b1.reshape(1, H),
      W2, b2.reshape(1, H))


def _stage2_body(o1, r0, r1, aW0, aW1, ht, hm, W1, b1, ps_ref):
    elu = _elu
    a0 = jnp.concatenate([o1[0, 0] + o1[0, 1], o1[1, 0] + o1[1, 1]],
                         axis=1) * r0[...]
    a1 = jnp.concatenate([o1[2, 0] + o1[2, 1], o1[3, 0] + o1[3, 1]],
                         axis=1) * r1[...]
    t0 = a0 @ aW0[...]
    t1 = a1 @ aW1[...]
    hs = [elu(ht[...] + t0), elu(hm[...] + t0), elu(ht[...] + t1),
          elu(hm[...] + t1), ht[...]]
    for v in range(5):
        ps_ref[v] = hs[v] @ W1[...] + b1[...]


def _stage2(out1, r0, r1, aW0, aW1, ht, hm, W1, b1):
    return pl.pallas_call(
        _stage2_body,
        grid=(NP // _BB,),
        in_specs=[
            pl.BlockSpec((4, NC, _BB, 64), lambda i: (0, 0, i, 0)),
            pl.BlockSpec((_BB, 1), lambda i: (i, 0)),
            pl.BlockSpec((_BB, 1), lambda i: (i, 0)),
            pl.BlockSpec((H, H), lambda i: (0, 0)),
            pl.BlockSpec((H, H), lambda i: (0, 0)),
            pl.BlockSpec((_BB, H), lambda i: (i, 0)),
            pl.BlockSpec((_BB, H), lambda i: (i, 0)),
            pl.BlockSpec((H, Z), lambda i: (0, 0)),
            pl.BlockSpec((1, Z), lambda i: (0, 0)),
        ],
        out_specs=pl.BlockSpec((5, _BB, Z), lambda i: (0, i, 0)),
        out_shape=jax.ShapeDtypeStruct((5, NP, Z), jnp.float32),
    )(out1, r0, r1, aW0, aW1, ht, hm, W1, b1.reshape(1, Z))


def _stage3_body(o2, ra, rb, rc, rd, re, W2, b2, qs_ref):
    rs = [ra, rb, rc, rd, re]
    for v in range(5):
        m = (o2[v, 0] + o2[v, 1]) * rs[v][...]
        qs_ref[v] = jax.nn.relu(m) @ W2[...] + b2[...]


def _stage3(out2, rcols, W2, b2):
    return pl.pallas_call(
        _stage3_body,
        grid=(NP // _BB,),
        in_specs=[
            pl.BlockSpec((5, NC, _BB, 64), lambda i: (0, 0, i, 0)),
        ] + [pl.BlockSpec((_BB, 1), lambda i: (i, 0))] * 5 + [
            pl.BlockSpec((Z, Z), lambda i: (0, 0)),
            pl.BlockSpec((1, Z), lambda i: (0, 0)),
        ],
        out_specs=pl.BlockSpec((5, _BB, Z), lambda i: (0, i, 0)),
        out_shape=jax.ShapeDtypeStruct((5, NP, Z), jnp.float32),
    )(out2, *rcols, W2, b2.reshape(1, Z))


def _stage4_body(o3, ra, rb, rc, rd, re, pW, pb, zn_ref, zc_ref):
    i = pl.program_id(0)
    rowid = i * _BB + lax.broadcasted_iota(jnp.int32, (_BB, 1), 0)
    valid = rowid < N
    rs = [ra, rb, rc, rd, re]
    for v in range(4):
        m = (o3[v, 0] + o3[v, 1]) * rs[v][...]
        nrm = jnp.sqrt(jnp.sum(m * m, axis=1, keepdims=True)) + 1e-8
        zn_ref[v] = jnp.where(valid, m / nrm, 0.0)
    m4 = (o3[4, 0] + o3[4, 1]) * re[...]
    zc = jnp.tanh(m4 @ pW[...] + pb[...])
    nrm = jnp.sqrt(jnp.sum(zc * zc, axis=1, keepdims=True)) + 1e-8
    zc_ref[...] = jnp.where(valid, zc / nrm, 0.0)


def _stage4(out3, rcols, pW, pb):
    return pl.pallas_call(
        _stage4_body,
        grid=(NP // _BB,),
        in_specs=[
            pl.BlockSpec((5, NC, _BB, 64), lambda i: (0, 0, i, 0)),
        ] + [pl.BlockSpec((_BB, 1), lambda i: (i, 0))] * 5 + [
            pl.BlockSpec((Z, Z), lambda i: (0, 0)),
            pl.BlockSpec((1, Z), lambda i: (0, 0)),
        ],
        out_specs=[
            pl.BlockSpec((4, _BB, Z), lambda i: (0, i, 0)),
            pl.BlockSpec((_BB, Z), lambda i: (i, 0)),
        ],
        out_shape=[
            jax.ShapeDtypeStruct((4, NP, Z), jnp.float32),
            jax.ShapeDtypeStruct((NP, Z), jnp.float32),
        ],
    )(out3, *rcols, pW, pb.reshape(1, Z))


def _stage5_body(zn, beta, pW, pb, zp_ref):
    i = pl.program_id(0)
    rowid = i * _BB + lax.broadcasted_iota(jnp.int32, (_BB, 1), 0)
    valid = rowid < N
    z = (zn[0] * beta[0:1, 0:1] + zn[1] * beta[1:2, 0:1] +
         zn[2] * beta[2:3, 0:1] + zn[3] * beta[3:4, 0:1])
    zp = jnp.tanh(z @ pW[...] + pb[...])
    nrm = jnp.sqrt(jnp.sum(zp * zp, axis=1, keepdims=True)) + 1e-8
    zp_ref[...] = jnp.where(valid, zp / nrm, 0.0)


def _stage5(zn, beta, pW, pb):
    return pl.pallas_call(
        _stage5_body,
        grid=(NP // _BB,),
        in_specs=[
            pl.BlockSpec((4, _BB, Z), lambda i: (0, i, 0)),
            pl.BlockSpec((4, 1), lambda i: (0, 0)),
            pl.BlockSpec((Z, Z), lambda i: (0, 0)),
            pl.BlockSpec((1, Z), lambda i: (0, 0)),
        ],
        out_specs=pl.BlockSpec((_BB, Z), lambda i: (i, 0)),
        out_shape=jax.ShapeDtypeStruct((NP, Z), jnp.float32),
    )(zn, beta.reshape(4, 1), pW, pb.reshape(1, Z))


_BM = 1024


def _contrast_body(z1_ref, z2_ref, row_ref, col_ref, diag_ref):
    i = pl.program_id(0)
    j = pl.program_id(1)
    s = lax.dot_general(z1_ref[...].astype(jnp.bfloat16),
                        z2_ref[...].astype(jnp.bfloat16),
                        (((1,), (1,)), ((), ())),
                        preferred_element_type=jnp.float32) * (1.0 / TAU)
    e = jnp.exp(s)
    rs = jnp.sum(e, axis=1)
    cs = jnp.sum(e, axis=0)

    @pl.when(j == 0)
    def _():
        row_ref[0, pl.ds(i * _BM, _BM)] = rs

    @pl.when(j != 0)
    def _():
        row_ref[0, pl.ds(i * _BM, _BM)] += rs

    @pl.when(i == 0)
    def _():
        col_ref[0, pl.ds(j * _BM, _BM)] = cs

    @pl.when(i != 0)
    def _():
        col_ref[0, pl.ds(j * _BM, _BM)] += cs

    @pl.when(i == j)
    def _():
        diag_ref[0, pl.ds(i * _BM, _BM)] = (
            jnp.sum(z1_ref[...] * z2_ref[...], axis=1) * (1.0 / TAU))


def _contrast(z1p, z2p):
    pad = NP - N
    grid = (NP // _BM, NP // _BM)
    row, col, diag = pl.pallas_call(
        _contrast_body,
        grid=grid,
        in_specs=[
            pl.BlockSpec((_BM, Z), lambda i, j: (i, 0)),
            pl.BlockSpec((_BM, Z), lambda i, j: (j, 0)),
        ],
        out_specs=[
            pl.BlockSpec((1, NP), lambda i, j: (0, 0)),
            pl.BlockSpec((1, NP), lambda i, j: (0, 0)),
            pl.BlockSpec((1, NP), lambda i, j: (0, 0)),
        ],
        out_shape=[
            jax.ShapeDtypeStruct((1, NP), jnp.float32),
            jax.ShapeDtypeStruct((1, NP), jnp.float32),
            jax.ShapeDtypeStruct((1, NP), jnp.float32),
        ],
    )(z1p, z2p)
    return row[0, :N] - pad, col[0, :N] - pad, diag[0, :N]


def _reshape_edges(edge, ch, g):
    per_w = E // NW
    return edge.reshape(2, NW, per_w // ch, ch // g, g)


def _l2norm(x):
    return x / (jnp.linalg.norm(x, axis=1, keepdims=True) + 1e-8)


def kernel(feats_0, feats_1, feats_2, mask_feat, fc_W0, fc_b0, fc_W1, fc_b1,
           fc_W2, fc_b2, agg_W0, agg_W1, gcn_W1, gcn_b1, gcn_W2, gcn_b2,
           proj_W, proj_b, att_W, att_b, att_a, nei_edge_0, nei_edge_1,
           mask_edge_0, mask_edge_1, norm_edge_0, norm_edge_1, adj_edge,
           num_cluster):
    edges = [nei_edge_0, nei_edge_1, mask_edge_0, mask_edge_1, norm_edge_0,
             norm_edge_1, adj_edge]
    cnt_ers = [e[0].reshape(NW, _CNT_NCH, 1, _CNT_CH) for e in edges]
    cparts = _get_count7()(*cnt_ers)
    cnt = cparts[:, :, 0, :].sum(axis=1)
    recip = 1.0 / jnp.maximum(cnt, 1.0)
    rcol = [recip[v][:, None] for v in range(7)]
    r_nei0, r_nei1, r_mask0, r_mask1, r_norm0, r_norm1, r_adj = rcol

    er_nei = [_reshape_edges(e, _SS_CH, _SS_G) for e in (nei_edge_0,
                                                         nei_edge_1)]
    er5 = [_reshape_edges(e, _SS_CH, _SS_G) for e in
           (norm_edge_0, mask_edge_0, norm_edge_1, mask_edge_1, adj_edge)]
    rs5 = [r_norm0, r_mask0, r_norm1, r_mask1, r_adj]

    pad = ((0, NP - N), (0, 0))
    h_tar, h_mask, hn0, hn1 = _encode(
        jnp.pad(feats_0, pad), jnp.pad(mask_feat, pad), jnp.pad(feats_1, pad),
        jnp.pad(feats_2, pad), fc_W0, fc_b0, fc_W1, fc_b1, fc_W2, fc_b2)

    x1 = jnp.stack([hn0[:, :64], hn0[:, 64:], hn1[:, :64], hn1[:, 64:]])
    out1 = _seg_sum_batch(x1, [er_nei[0], er_nei[0], er_nei[1], er_nei[1]])

    ps = _stage2(out1, r_nei0, r_nei1, agg_W0, agg_W1, h_tar, h_mask,
                 gcn_W1, gcn_b1)

    out2 = _seg_sum_batch(ps, er5)
    qs = _stage3(out2, rs5, gcn_W2, gcn_b2)

    out3 = _seg_sum_batch(qs, er5)
    zn, z_coarse = _stage4(out3, rs5, proj_W, proj_b)

    zs = zn[:, :N]
    w = (jnp.tanh(zs @ att_W + att_b) @ att_a).mean(axis=1)
    beta = jax.nn.softmax(w)
    z_pro = _stage5(zn, beta, proj_W, proj_b)

    row_se, col_se, diag = _contrast(z_coarse, z_pro)
    l1 = (jnp.log(row_se) - diag).mean()
    l2_ = (jnp.log(col_se) - diag).mean()
    loss_info = 0.5 * (l1 + l2_)

    zp = z_pro[:N]
    assign = jnp.arange(N) % NUM_CLUSTER
    protos = _l2norm(zp.reshape(N // NUM_CLUSTER, NUM_CLUSTER, Z).mean(0))
    logits = zp @ protos.T / TAU
    pos = jnp.take_along_axis(logits, assign[:, None], axis=1)[:, 0]
    loss_proto = (jax.nn.logsumexp(logits, axis=1) - pos).mean()
    return loss_info + loss_proto


# speedup vs baseline: 1.1180x; 1.0344x over previous
import functools

import jax
import jax.numpy as jnp
from jax import lax
from jax.experimental import pallas as pl
from jax.experimental.pallas import tpu as pltpu
from jax.experimental.pallas import tpu_sc as plsc

N = 10000
E = 320000
D = 128
H = 128
Z = 64
TAU = 0.8
NUM_CLUSTER = 20

NC = 2
NS = 16
NW = NC * NS
NP = 10240
RPT = NP // NS


def _seg_sum_multi_kernel(k, ch, g):
    per_w = E // NW
    n_ch = per_w // ch
    n_g = ch // g
    zr = min(ch - ch % 8, RPT)
    assert n_ch % 2 == 0
    mesh = plsc.VectorSubcoreMesh(core_axis_name="c", subcore_axis_name="s",
                                  num_cores=NC, num_subcores=NS)

    @functools.partial(
        pl.kernel,
        out_type=jax.ShapeDtypeStruct((k, NC, NP, 64), jnp.float32),
        mesh=mesh,
        compiler_params=pltpu.CompilerParams(use_tc_tiling_on_sc=False),
        scratch_types=[
            pltpu.VMEM((n_ch, n_g, g), jnp.int32),
            pltpu.VMEM((n_ch, n_g, g), jnp.int32),
            pltpu.VMEM((ch, 64), jnp.float32),
            pltpu.VMEM((ch, 64), jnp.float32),
            pltpu.VMEM_SHARED((NP, 64), jnp.float32),
            pltpu.SemaphoreType.DMA((ch // g,)),
            pltpu.SemaphoreType.DMA,
        ],
    )
    def body(*refs):
        x_stk = refs[0]
        ers = refs[1:1 + k]
        zero_hbm = refs[1 + k]
        out_hbm = refs[2 + k]
        dst_v, src_v, rows_a, rows_b, acc_sh, sem_g, sem_s = refs[3 + k:]
        cid = lax.axis_index("c")
        sid = lax.axis_index("s")
        wid = sid * NC + cid

        for p in range(k):
            x_hbm, er_hbm = x_stk.at[p], ers[p]
            pltpu.sync_copy(er_hbm.at[0, wid], dst_v)
            pltpu.sync_copy(er_hbm.at[1, wid], src_v)
            pltpu.sync_copy(zero_hbm, rows_a.at[pl.ds(0, zr)])
            for o in range(0, RPT, zr):
                m = min(zr, RPT - o)
                pltpu.sync_copy(rows_a.at[pl.ds(0, m)],
                                acc_sh.at[pl.ds(sid * RPT + o, m)])
            plsc.subcore_barrier()

            def pair(t, carry, x_hbm=x_hbm):
                a = 2 * t
                b = a + 1
                ga = [
                    pltpu.async_copy(x_hbm.at[src_v.at[a, i]],
                                     rows_a.at[pl.ds(i * g, g)], sem_g.at[i])
                    for i in range(n_g)
                ]
                sa, gb, sb = [], [], []
                for i in range(n_g):
                    ga[i].wait()
                    sa.append(
                        pltpu.async_copy(rows_a.at[pl.ds(i * g, g)],
                                         acc_sh.at[dst_v.at[a, i]], sem_s,
                                         add=True))
                    gb.append(
                        pltpu.async_copy(x_hbm.at[src_v.at[b, i]],
                                         rows_b.at[pl.ds(i * g, g)],
                                         sem_g.at[i]))
                for i in range(n_g):
                    gb[i].wait()
                    sb.append(
                        pltpu.async_copy(rows_b.at[pl.ds(i * g, g)],
                                         acc_sh.at[dst_v.at[b, i]], sem_s,
                                         add=True))
                for de in sa + sb:
                    de.wait()
                return carry

            lax.fori_loop(0, n_ch // 2, pair, 0)
            plsc.subcore_barrier()

            for o in range(0, RPT, zr):
                m = min(zr, RPT - o)
                pltpu.sync_copy(acc_sh.at[pl.ds(sid * RPT + o, m)],
                                rows_a.at[pl.ds(0, m)])
                pltpu.sync_copy(rows_a.at[pl.ds(0, m)],
                                out_hbm.at[p, cid, pl.ds(sid * RPT + o, m)])
            if p + 1 < k:
                plsc.subcore_barrier()

    return body


_SS_CH = 500
_SS_G = 100


@functools.lru_cache(maxsize=None)
def _get_seg_sum_multi(k):
    return _seg_sum_multi_kernel(k, _SS_CH, _SS_G)


def _seg_sum_batch(x_stk, ers):
    zeros = jnp.zeros((min(_SS_CH - _SS_CH % 8, RPT), 64), jnp.float32)
    return _get_seg_sum_multi(len(ers))(x_stk, *ers, zeros)


_CNT_CH = 2000
_CNT_NCH = (E // NW) // _CNT_CH


def _make_count_kernel(n_lists):
    mesh = plsc.VectorSubcoreMesh(core_axis_name="c", subcore_axis_name="s",
                                  num_cores=NC, num_subcores=NS)

    @functools.partial(
        pl.kernel,
        out_type=jax.ShapeDtypeStruct((n_lists, NW, 1, NP), jnp.float32),
        mesh=mesh,
        compiler_params=pltpu.CompilerParams(use_tc_tiling_on_sc=False,
                                             needs_layout_passes=False),
        scratch_types=[
            pltpu.VMEM((_CNT_CH,), jnp.int32),
            pltpu.VMEM((NP,), jnp.float32),
        ],
    )
    def body(*refs):
        ers = refs[:n_lists]
        out_hbm = refs[n_lists]
        didx, cnt_v = refs[n_lists + 1:]
        cid = lax.axis_index("c")
        sid = lax.axis_index("s")
        wid = sid * NC + cid
        ones = jnp.ones((16,), jnp.float32)
        zeros = jnp.zeros((16,), jnp.float32)

        for l in range(n_lists):
            def zero(v, carry):
                cnt_v[pl.ds(v * 16, 16)] = zeros
                return carry
            lax.fori_loop(0, NP // 16, zero, 0)

            def chunk(c, carry, er=ers[l]):
                pltpu.sync_copy(er.at[wid, c, 0], didx)

                def group(v, carry2):
                    idx = didx[pl.ds(v * 16, 16)]
                    plsc.addupdate_scatter(cnt_v, [idx], ones)
                    return carry2

                lax.fori_loop(0, _CNT_CH // 16, group, 0)
                return carry

            lax.fori_loop(0, _CNT_NCH, chunk, 0)
            pltpu.sync_copy(cnt_v, out_hbm.at[l, wid, 0])

    return body


_get_count7 = functools.lru_cache(maxsize=None)(lambda: _make_count_kernel(7))


_BB = 2048


def _elu(x):
    return jnp.where(x > 0, x, jnp.exp(x) - 1.0)


def _encode_body(f0, fm, f1, f2, W0, b0, W1, b1, W2, b2,
                 ht_ref, hm_ref, hn0_ref, hn1_ref):
    elu = _elu
    ht_ref[...] = elu(f0[...] @ W0[...] + b0[...])
    hm_ref[...] = elu(fm[...] @ W0[...] + b0[...])
    hn0_ref[...] = elu(f1[...] @ W1[...] + b1[...])
    hn1_ref[...] = elu(f2[...] @ W2[...] + b2[...])


def _encode(f0p, fmp, f1p, f2p, W0, b0, W1, b1, W2, b2):
    bs_x = pl.BlockSpec((_BB, D), lambda i: (i, 0))
    bs_w = pl.BlockSpec((D, H), lambda i: (0, 0))
    bs_b = pl.BlockSpec((1, H), lambda i: (0, 0))
    return pl.pallas_call(
        _encode_body,
        grid=(NP // _BB,),
        in_specs=[bs_x, bs_x, bs_x, bs_x, bs_w, bs_b, bs_w, bs_b, bs_w, bs_b],
        out_specs=[pl.BlockSpec((_BB, H), lambda i: (i, 0))] * 4,
        out_shape=[jax.ShapeDtypeStruct((NP, H), jnp.float32)] * 4,
    )(f0p, fmp, f1p, f2p, W0, b0.reshape(1, H), W1, b1.reshape(1, H),
      W2, b2.reshape(1, H))


def _stage2_body(o1, r0, r1, aW0, aW1, ht, hm, W1, b1, ps_ref):
    elu = _elu
    a0 = jnp.concatenate([o1[0, 0] + o1[0, 1], o1[1, 0] + o1[1, 1]],
                         axis=1) * r0[...]
    a1 = jnp.concatenate([o1[2, 0] + o1[2, 1], o1[3, 0] + o1[3, 1]],
                         axis=1) * r1[...]
    t0 = a0 @ aW0[...]
    t1 = a1 @ aW1[...]
    hs = [elu(ht[...] + t0), elu(hm[...] + t0), elu(ht[...] + t1),
          elu(hm[...] + t1), ht[...]]
    for v in range(5):
        ps_ref[v] = hs[v] @ W1[...] + b1[...]


def _stage2(out1, r0, r1, aW0, aW1, ht, hm, W1, b1):
    return pl.pallas_call(
        _stage2_body,
        grid=(NP // _BB,),
        in_specs=[
            pl.BlockSpec((4, NC, _BB, 64), lambda i: (0, 0, i, 0)),
            pl.BlockSpec((_BB, 1), lambda i: (i, 0)),
            pl.BlockSpec((_BB, 1), lambda i: (i, 0)),
            pl.BlockSpec((H, H), lambda i: (0, 0)),
            pl.BlockSpec((H, H), lambda i: (0, 0)),
            pl.BlockSpec((_BB, H), lambda i: (i, 0)),
            pl.BlockSpec((_BB, H), lambda i: (i, 0)),
            pl.BlockSpec((H, Z), lambda i: (0, 0)),
            pl.BlockSpec((1, Z), lambda i: (0, 0)),
        ],
        out_specs=pl.BlockSpec((5, _BB, Z), lambda i: (0, i, 0)),
        out_shape=jax.ShapeDtypeStruct((5, NP, Z), jnp.float32),
    )(out1, r0, r1, aW0, aW1, ht, hm, W1, b1.reshape(1, Z))


def _stage3_body(o2, ra, rb, rc, rd, re, W2, b2, qs_ref):
    rs = [ra, rb, rc, rd, re]
    for v in range(5):
        m = (o2[v, 0] + o2[v, 1]) * rs[v][...]
        qs_ref[v] = jax.nn.relu(m) @ W2[...] + b2[...]


def _stage3(out2, rcols, W2, b2):
    return pl.pallas_call(
        _stage3_body,
        grid=(NP // _BB,),
        in_specs=[
            pl.BlockSpec((5, NC, _BB, 64), lambda i: (0, 0, i, 0)),
        ] + [pl.BlockSpec((_BB, 1), lambda i: (i, 0))] * 5 + [
            pl.BlockSpec((Z, Z), lambda i: (0, 0)),
            pl.BlockSpec((1, Z), lambda i: (0, 0)),
        ],
        out_specs=pl.BlockSpec((5, _BB, Z), lambda i: (0, i, 0)),
        out_shape=jax.ShapeDtypeStruct((5, NP, Z), jnp.float32),
    )(out2, *rcols, W2, b2.reshape(1, Z))


def _stage4_body(o3, ra, rb, rc, rd, re, pW, pb, zn_ref, zc_ref):
    i = pl.program_id(0)
    rowid = i * _BB + lax.broadcasted_iota(jnp.int32, (_BB, 1), 0)
    valid = rowid < N
    rs = [ra, rb, rc, rd, re]
    for v in range(4):
        m = (o3[v, 0] + o3[v, 1]) * rs[v][...]
        nrm = jnp.sqrt(jnp.sum(m * m, axis=1, keepdims=True)) + 1e-8
        zn_ref[v] = jnp.where(valid, m / nrm, 0.0)
    m4 = (o3[4, 0] + o3[4, 1]) * re[...]
    zc = jnp.tanh(m4 @ pW[...] + pb[...])
    nrm = jnp.sqrt(jnp.sum(zc * zc, axis=1, keepdims=True)) + 1e-8
    zc_ref[...] = jnp.where(valid, zc / nrm, 0.0)


def _stage4(out3, rcols, pW, pb):
    return pl.pallas_call(
        _stage4_body,
        grid=(NP // _BB,),
        in_specs=[
            pl.BlockSpec((5, NC, _BB, 64), lambda i: (0, 0, i, 0)),
        ] + [pl.BlockSpec((_BB, 1), lambda i: (i, 0))] * 5 + [
            pl.BlockSpec((Z, Z), lambda i: (0, 0)),
            pl.BlockSpec((1, Z), lambda i: (0, 0)),
        ],
        out_specs=[
            pl.BlockSpec((4, _BB, Z), lambda i: (0, i, 0)),
            pl.BlockSpec((_BB, Z), lambda i: (i, 0)),
        ],
        out_shape=[
            jax.ShapeDtypeStruct((4, NP, Z), jnp.float32),
            jax.ShapeDtypeStruct((NP, Z), jnp.float32),
        ],
    )(out3, *rcols, pW, pb.reshape(1, Z))


def _stage5_body(zn, beta, pW, pb, zp_ref):
    i = pl.program_id(0)
    rowid = i * _BB + lax.broadcasted_iota(jnp.int32, (_BB, 1), 0)
    valid = rowid < N
    z = (zn[0] * beta[0:1, 0:1] + zn[1] * beta[1:2, 0:1] +
         zn[2] * beta[2:3, 0:1] + zn[3] * beta[3:4, 0:1])
    zp = jnp.tanh(z @ pW[...] + pb[...])
    nrm = jnp.sqrt(jnp.sum(zp * zp, axis=1, keepdims=True)) + 1e-8
    zp_ref[...] = jnp.where(valid, zp / nrm, 0.0)


def _stage5(zn, beta, pW, pb):
    return pl.pallas_call(
        _stage5_body,
        grid=(NP // _BB,),
        in_specs=[
            pl.BlockSpec((4, _BB, Z), lambda i: (0, i, 0)),
            pl.BlockSpec((4, 1), lambda i: (0, 0)),
            pl.BlockSpec((Z, Z), lambda i: (0, 0)),
            pl.BlockSpec((1, Z), lambda i: (0, 0)),
        ],
        out_specs=pl.BlockSpec((_BB, Z), lambda i: (i, 0)),
        out_shape=jax.ShapeDtypeStruct((NP, Z), jnp.float32),
    )(zn, beta.reshape(4, 1), pW, pb.reshape(1, Z))


_BM = 1024


def _contrast_body(z1_ref, z2_ref, row_ref, col_ref, diag_ref):
    i = pl.program_id(0)
    j = pl.program_id(1)
    s = lax.dot_general(z1_ref[...].astype(jnp.bfloat16),
                        z2_ref[...].astype(jnp.bfloat16),
                        (((1,), (1,)), ((), ())),
                        preferred_element_type=jnp.float32) * (1.0 / TAU)
    e = jnp.exp(s)
    rs = jnp.sum(e, axis=1)
    cs = jnp.sum(e, axis=0)

    @pl.when(j == 0)
    def _():
        row_ref[0, pl.ds(i * _BM, _BM)] = rs

    @pl.when(j != 0)
    def _():
        row_ref[0, pl.ds(i * _BM, _BM)] += rs

    @pl.when(i == 0)
    def _():
        col_ref[0, pl.ds(j * _BM, _BM)] = cs

    @pl.when(i != 0)
    def _():
        col_ref[0, pl.ds(j * _BM, _BM)] += cs

    @pl.when(i == j)
    def _():
        diag_ref[0, pl.ds(i * _BM, _BM)] = (
            jnp.sum(z1_ref[...] * z2_ref[...], axis=1) * (1.0 / TAU))


def _contrast(z1p, z2p):
    pad = NP - N
    grid = (NP // _BM, NP // _BM)
    row, col, diag = pl.pallas_call(
        _contrast_body,
        grid=grid,
        in_specs=[
            pl.BlockSpec((_BM, Z), lambda i, j: (i, 0)),
            pl.BlockSpec((_BM, Z), lambda i, j: (j, 0)),
        ],
        out_specs=[
            pl.BlockSpec((1, NP), lambda i, j: (0, 0)),
            pl.BlockSpec((1, NP), lambda i, j: (0, 0)),
            pl.BlockSpec((1, NP), lambda i, j: (0, 0)),
        ],
        out_shape=[
            jax.ShapeDtypeStruct((1, NP), jnp.float32),
            jax.ShapeDtypeStruct((1, NP), jnp.float32),
            jax.ShapeDtypeStruct((1, NP), jnp.float32),
        ],
    )(z1p, z2p)
    return row[0, :N] - pad, col[0, :N] - pad, diag[0, :N]


def _reshape_edges(edge, ch, g):
    per_w = E // NW
    return edge.reshape(2, NW, per_w // ch, ch // g, g)


def _l2norm(x):
    return x / (jnp.linalg.norm(x, axis=1, keepdims=True) + 1e-8)


def kernel(feats_0, feats_1, feats_2, mask_feat, fc_W0, fc_b0, fc_W1, fc_b1,
           fc_W2, fc_b2, agg_W0, agg_W1, gcn_W1, gcn_b1, gcn_W2, gcn_b2,
           proj_W, proj_b, att_W, att_b, att_a, nei_edge_0, nei_edge_1,
           mask_edge_0, mask_edge_1, norm_edge_0, norm_edge_1, adj_edge,
           num_cluster):
    edges = [nei_edge_0, nei_edge_1, mask_edge_0, mask_edge_1, norm_edge_0,
             norm_edge_1, adj_edge]
    cnt_ers = [e[0].reshape(NW, _CNT_NCH, 1, _CNT_CH) for e in edges]
    cparts = _get_count7()(*cnt_ers)
    cnt = cparts[:, :, 0, :].sum(axis=1)
    recip = 1.0 / jnp.maximum(cnt, 1.0)
    rcol = [recip[v][:, None] for v in range(7)]
    r_nei0, r_nei1, r_mask0, r_mask1, r_norm0, r_norm1, r_adj = rcol

    er_nei = [_reshape_edges(e, _SS_CH, _SS_G) for e in (nei_edge_0,
                                                         nei_edge_1)]
    er5 = [_reshape_edges(e, _SS_CH, _SS_G) for e in
           (norm_edge_0, mask_edge_0, norm_edge_1, mask_edge_1, adj_edge)]
    rs5 = [r_norm0, r_mask0, r_norm1, r_mask1, r_adj]

    pad = ((0, NP - N), (0, 0))
    h_tar, h_mask, hn0, hn1 = _encode(
        jnp.pad(feats_0, pad), jnp.pad(mask_feat, pad), jnp.pad(feats_1, pad),
        jnp.pad(feats_2, pad), fc_W0, fc_b0, fc_W1, fc_b1, fc_W2, fc_b2)

    x1 = jnp.stack([hn0[:, :64], hn0[:, 64:], hn1[:, :64], hn1[:, 64:]])
    out1 = _seg_sum_batch(x1, [er_nei[0], er_nei[0], er_nei[1], er_nei[1]])

    ps = _stage2(out1, r_nei0, r_nei1, agg_W0, agg_W1, h_tar, h_mask,
                 gcn_W1, gcn_b1)

    out2 = _seg_sum_batch(ps, er5)
    qs = _stage3(out2, rs5, gcn_W2, gcn_b2)

    out3 = _seg_sum_batch(qs, er5)
    zn, z_coarse = _stage4(out3, rs5, proj_W, proj_b)

    zs = zn[:, :N]
    w = (jnp.tanh(zs @ att_W + att_b) @ att_a).mean(axis=1)
    beta = jax.nn.softmax(w)
    z_pro = _stage5(zn, beta, proj_W, proj_b)

    row_se, col_se, diag = _contrast(z_coarse, z_pro)
    l1 = (jnp.log(row_se) - diag).mean()
    l2_ = (jnp.log(col_se) - diag).mean()
    loss_info = 0.5 * (l1 + l2_)

    zp = z_pro[:N]
    assign = jnp.arange(N) % NUM_CLUSTER
    protos = _l2norm(zp.reshape(N // NUM_CLUSTER, NUM_CLUSTER, Z).mean(0))
    logits = zp @ protos.T / TAU
    pos = jnp.take_along_axis(logits, assign[:, None], axis=1)[:, 0]
    loss_proto = (jax.nn.logsumexp(logits, axis=1) - pos).mean()
    return loss_info + loss_proto


# speedup vs baseline: 1.1182x; 1.0002x over previous
import functools

import jax
import jax.numpy as jnp
from jax import lax
from jax.experimental import pallas as pl
from jax.experimental.pallas import tpu as pltpu
from jax.experimental.pallas import tpu_sc as plsc

N = 10000
E = 320000
D = 128
H = 128
Z = 64
TAU = 0.8
NUM_CLUSTER = 20

NC = 2
NS = 16
NW = NC * NS
NP = 10240
RPT = NP // NS


def _seg_sum_multi_kernel(k, ch, g):
    per_w = E // NW
    n_ch = per_w // ch
    n_g = ch // g
    zr = min(ch - ch % 8, RPT)
    assert n_ch % 2 == 0
    mesh = plsc.VectorSubcoreMesh(core_axis_name="c", subcore_axis_name="s",
                                  num_cores=NC, num_subcores=NS)

    @functools.partial(
        pl.kernel,
        out_type=jax.ShapeDtypeStruct((k, NC, NP, 64), jnp.float32),
        mesh=mesh,
        compiler_params=pltpu.CompilerParams(use_tc_tiling_on_sc=False),
        scratch_types=[
            pltpu.VMEM((n_ch, n_g, g), jnp.int32),
            pltpu.VMEM((n_ch, n_g, g), jnp.int32),
            pltpu.VMEM((ch, 64), jnp.float32),
            pltpu.VMEM((ch, 64), jnp.float32),
            pltpu.VMEM_SHARED((NP, 64), jnp.float32),
            pltpu.SemaphoreType.DMA((ch // g,)),
            pltpu.SemaphoreType.DMA,
        ],
    )
    def body(*refs):
        x_stk = refs[0]
        ers = refs[1:1 + k]
        zero_hbm = refs[1 + k]
        out_hbm = refs[2 + k]
        dst_v, src_v, rows_a, rows_b, acc_sh, sem_g, sem_s = refs[3 + k:]
        cid = lax.axis_index("c")
        sid = lax.axis_index("s")
        wid = sid * NC + cid

        for p in range(k):
            x_hbm, er_hbm = x_stk.at[p], ers[p]
            pltpu.sync_copy(er_hbm.at[0, wid], dst_v)
            pltpu.sync_copy(er_hbm.at[1, wid], src_v)
            pltpu.sync_copy(zero_hbm, rows_a.at[pl.ds(0, zr)])
            for o in range(0, RPT, zr):
                m = min(zr, RPT - o)
                pltpu.sync_copy(rows_a.at[pl.ds(0, m)],
                                acc_sh.at[pl.ds(sid * RPT + o, m)])
            plsc.subcore_barrier()

            def pair(t, carry, x_hbm=x_hbm):
                a = 2 * t
                b = a + 1

                @pl.when(t > 0)
                def _():
                    for i in range(2 * n_g):
                        pltpu.make_async_copy(
                            rows_a.at[pl.ds(0, g)],
                            acc_sh.at[pl.ds(0, g)], sem_s).wait()

                ga = [
                    pltpu.async_copy(x_hbm.at[src_v.at[a, i]],
                                     rows_a.at[pl.ds(i * g, g)], sem_g.at[i])
                    for i in range(n_g)
                ]
                sa, gb, sb = [], [], []
                for i in range(n_g):
                    ga[i].wait()
                    sa.append(
                        pltpu.async_copy(rows_a.at[pl.ds(i * g, g)],
                                         acc_sh.at[dst_v.at[a, i]], sem_s,
                                         add=True))
                    gb.append(
                        pltpu.async_copy(x_hbm.at[src_v.at[b, i]],
                                         rows_b.at[pl.ds(i * g, g)],
                                         sem_g.at[i]))
                for i in range(n_g):
                    gb[i].wait()
                    sb.append(
                        pltpu.async_copy(rows_b.at[pl.ds(i * g, g)],
                                         acc_sh.at[dst_v.at[b, i]], sem_s,
                                         add=True))
                return carry

            lax.fori_loop(0, n_ch // 2, pair, 0)
            for i in range(2 * n_g):
                pltpu.make_async_copy(rows_a.at[pl.ds(0, g)],
                                      acc_sh.at[pl.ds(0, g)], sem_s).wait()
            plsc.subcore_barrier()

            for o in range(0, RPT, zr):
                m = min(zr, RPT - o)
                pltpu.sync_copy(acc_sh.at[pl.ds(sid * RPT + o, m)],
                                rows_a.at[pl.ds(0, m)])
                pltpu.sync_copy(rows_a.at[pl.ds(0, m)],
                                out_hbm.at[p, cid, pl.ds(sid * RPT + o, m)])
            if p + 1 < k:
                plsc.subcore_barrier()

    return body


_SS_CH = 500
_SS_G = 100


@functools.lru_cache(maxsize=None)
def _get_seg_sum_multi(k):
    return _seg_sum_multi_kernel(k, _SS_CH, _SS_G)


def _seg_sum_batch(x_stk, ers):
    zeros = jnp.zeros((min(_SS_CH - _SS_CH % 8, RPT), 64), jnp.float32)
    return _get_seg_sum_multi(len(ers))(x_stk, *ers, zeros)


_CNT_CH = 2000
_CNT_NCH = (E // NW) // _CNT_CH


def _make_count_kernel(n_lists):
    mesh = plsc.VectorSubcoreMesh(core_axis_name="c", subcore_axis_name="s",
                                  num_cores=NC, num_subcores=NS)

    @functools.partial(
        pl.kernel,
        out_type=jax.ShapeDtypeStruct((n_lists, NW, 1, NP), jnp.float32),
        mesh=mesh,
        compiler_params=pltpu.CompilerParams(use_tc_tiling_on_sc=False,
                                             needs_layout_passes=False),
        scratch_types=[
            pltpu.VMEM((_CNT_CH,), jnp.int32),
            pltpu.VMEM((NP,), jnp.float32),
        ],
    )
    def body(*refs):
        ers = refs[:n_lists]
        out_hbm = refs[n_lists]
        didx, cnt_v = refs[n_lists + 1:]
        cid = lax.axis_index("c")
        sid = lax.axis_index("s")
        wid = sid * NC + cid
        ones = jnp.ones((16,), jnp.float32)
        zeros = jnp.zeros((16,), jnp.float32)

        for l in range(n_lists):
            def zero(v, carry):
                cnt_v[pl.ds(v * 16, 16)] = zeros
                return carry
            lax.fori_loop(0, NP // 16, zero, 0)

            def chunk(c, carry, er=ers[l]):
                pltpu.sync_copy(er.at[wid, c, 0], didx)

                def group(v, carry2):
                    idx = didx[pl.ds(v * 16, 16)]
                    plsc.addupdate_scatter(cnt_v, [idx], ones)
                    return carry2

                lax.fori_loop(0, _CNT_CH // 16, group, 0)
                return carry

            lax.fori_loop(0, _CNT_NCH, chunk, 0)
            pltpu.sync_copy(cnt_v, out_hbm.at[l, wid, 0])

    return body


_get_count7 = functools.lru_cache(maxsize=None)(lambda: _make_count_kernel(7))


_BB = 2048


def _elu(x):
    return jnp.where(x > 0, x, jnp.exp(x) - 1.0)


def _encode_body(f0, fm, f1, f2, W0, b0, W1, b1, W2, b2,
                 ht_ref, hm_ref, hn0_ref, hn1_ref):
    elu = _elu
    ht_ref[...] = elu(f0[...] @ W0[...] + b0[...])
    hm_ref[...] = elu(fm[...] @ W0[...] + b0[...])
    hn0_ref[...] = elu(f1[...] @ W1[...] + b1[...])
    hn1_ref[...] = elu(f2[...] @ W2[...] + b2[...])


def _encode(f0p, fmp, f1p, f2p, W0, b0, W1, b1, W2, b2):
    bs_x = pl.BlockSpec((_BB, D), lambda i: (i, 0))
    bs_w = pl.BlockSpec((D, H), lambda i: (0, 0))
    bs_b = pl.BlockSpec((1, H), lambda i: (0, 0))
    return pl.pallas_call(
        _encode_body,
        grid=(NP // _BB,),
        in_specs=[bs_x, bs_x, bs_x, bs_x, bs_w, bs_b, bs_w, bs_b, bs_w, bs_b],
        out_specs=[pl.BlockSpec((_BB, H), lambda i: (i, 0))] * 4,
        out_shape=[jax.ShapeDtypeStruct((NP, H), jnp.float32)] * 4,
    )(f0p, fmp, f1p, f2p, W0, b0.reshape(1, H), W1, b1.reshape(1, H),
      W2, b2.reshape(1, H))


def _stage2_body(o1, r0, r1, aW0, aW1, ht, hm, W1, b1, ps_ref):
    elu = _elu
    a0 = jnp.concatenate([o1[0, 0] + o1[0, 1], o1[1, 0] + o1[1, 1]],
                         axis=1) * r0[...]
    a1 = jnp.concatenate([o1[2, 0] + o1[2, 1], o1[3, 0] + o1[3, 1]],
                         axis=1) * r1[...]
    t0 = a0 @ aW0[...]
    t1 = a1 @ aW1[...]
    hs = [elu(ht[...] + t0), elu(hm[...] + t0), elu(ht[...] + t1),
          elu(hm[...] + t1), ht[...]]
    for v in range(5):
        ps_ref[v] = hs[v] @ W1[...] + b1[...]


def _stage2(out1, r0, r1, aW0, aW1, ht, hm, W1, b1):
    return pl.pallas_call(
        _stage2_body,
        grid=(NP // _BB,),
        in_specs=[
            pl.BlockSpec((4, NC, _BB, 64), lambda i: (0, 0, i, 0)),
            pl.BlockSpec((_BB, 1), lambda i: (i, 0)),
            pl.BlockSpec((_BB, 1), lambda i: (i, 0)),
            pl.BlockSpec((H, H), lambda i: (0, 0)),
            pl.BlockSpec((H, H), lambda i: (0, 0)),
            pl.BlockSpec((_BB, H), lambda i: (i, 0)),
            pl.BlockSpec((_BB, H), lambda i: (i, 0)),
            pl.BlockSpec((H, Z), lambda i: (0, 0)),
            pl.BlockSpec((1, Z), lambda i: (0, 0)),
        ],
        out_specs=pl.BlockSpec((5, _BB, Z), lambda i: (0, i, 0)),
        out_shape=jax.ShapeDtypeStruct((5, NP, Z), jnp.float32),
    )(out1, r0, r1, aW0, aW1, ht, hm, W1, b1.reshape(1, Z))


def _stage3_body(o2, ra, rb, rc, rd, re, W2, b2, qs_ref):
    rs = [ra, rb, rc, rd, re]
    for v in range(5):
        m = (o2[v, 0] + o2[v, 1]) * rs[v][...]
        qs_ref[v] = jax.nn.relu(m) @ W2[...] + b2[...]


def _stage3(out2, rcols, W2, b2):
    return pl.pallas_call(
        _stage3_body,
        grid=(NP // _BB,),
        in_specs=[
            pl.BlockSpec((5, NC, _BB, 64), lambda i: (0, 0, i, 0)),
        ] + [pl.BlockSpec((_BB, 1), lambda i: (i, 0))] * 5 + [
            pl.BlockSpec((Z, Z), lambda i: (0, 0)),
            pl.BlockSpec((1, Z), lambda i: (0, 0)),
        ],
        out_specs=pl.BlockSpec((5, _BB, Z), lambda i: (0, i, 0)),
        out_shape=jax.ShapeDtypeStruct((5, NP, Z), jnp.float32),
    )(out2, *rcols, W2, b2.reshape(1, Z))


def _stage4_body(o3, ra, rb, rc, rd, re, pW, pb, zn_ref, zc_ref):
    i = pl.program_id(0)
    rowid = i * _BB + lax.broadcasted_iota(jnp.int32, (_BB, 1), 0)
    valid = rowid < N
    rs = [ra, rb, rc, rd, re]
    for v in range(4):
        m = (o3[v, 0] + o3[v, 1]) * rs[v][...]
        nrm = jnp.sqrt(jnp.sum(m * m, axis=1, keepdims=True)) + 1e-8
        zn_ref[v] = jnp.where(valid, m / nrm, 0.0)
    m4 = (o3[4, 0] + o3[4, 1]) * re[...]
    zc = jnp.tanh(m4 @ pW[...] + pb[...])
    nrm = jnp.sqrt(jnp.sum(zc * zc, axis=1, keepdims=True)) + 1e-8
    zc_ref[...] = jnp.where(valid, zc / nrm, 0.0)


def _stage4(out3, rcols, pW, pb):
    return pl.pallas_call(
        _stage4_body,
        grid=(NP // _BB,),
        in_specs=[
            pl.BlockSpec((5, NC, _BB, 64), lambda i: (0, 0, i, 0)),
        ] + [pl.BlockSpec((_BB, 1), lambda i: (i, 0))] * 5 + [
            pl.BlockSpec((Z, Z), lambda i: (0, 0)),
            pl.BlockSpec((1, Z), lambda i: (0, 0)),
        ],
        out_specs=[
            pl.BlockSpec((4, _BB, Z), lambda i: (0, i, 0)),
            pl.BlockSpec((_BB, Z), lambda i: (i, 0)),
        ],
        out_shape=[
            jax.ShapeDtypeStruct((4, NP, Z), jnp.float32),
            jax.ShapeDtypeStruct((NP, Z), jnp.float32),
        ],
    )(out3, *rcols, pW, pb.reshape(1, Z))


def _stage5_body(zn, beta, pW, pb, zp_ref):
    i = pl.program_id(0)
    rowid = i * _BB + lax.broadcasted_iota(jnp.int32, (_BB, 1), 0)
    valid = rowid < N
    z = (zn[0] * beta[0:1, 0:1] + zn[1] * beta[1:2, 0:1] +
         zn[2] * beta[2:3, 0:1] + zn[3] * beta[3:4, 0:1])
    zp = jnp.tanh(z @ pW[...] + pb[...])
    nrm = jnp.sqrt(jnp.sum(zp * zp, axis=1, keepdims=True)) + 1e-8
    zp_ref[...] = jnp.where(valid, zp / nrm, 0.0)


def _stage5(zn, beta, pW, pb):
    return pl.pallas_call(
        _stage5_body,
        grid=(NP // _BB,),
        in_specs=[
            pl.BlockSpec((4, _BB, Z), lambda i: (0, i, 0)),
            pl.BlockSpec((4, 1), lambda i: (0, 0)),
            pl.BlockSpec((Z, Z), lambda i: (0, 0)),
            pl.BlockSpec((1, Z), lambda i: (0, 0)),
        ],
        out_specs=pl.BlockSpec((_BB, Z), lambda i: (i, 0)),
        out_shape=jax.ShapeDtypeStruct((NP, Z), jnp.float32),
    )(zn, beta.reshape(4, 1), pW, pb.reshape(1, Z))


_BM = 1024


def _contrast_body(z1_ref, z2_ref, row_ref, col_ref, diag_ref):
    i = pl.program_id(0)
    j = pl.program_id(1)
    s = lax.dot_general(z1_ref[...].astype(jnp.bfloat16),
                        z2_ref[...].astype(jnp.bfloat16),
                        (((1,), (1,)), ((), ())),
                        preferred_element_type=jnp.float32) * (1.0 / TAU)
    e = jnp.exp(s)
    rs = jnp.sum(e, axis=1)
    cs = jnp.sum(e, axis=0)

    @pl.when(j == 0)
    def _():
        row_ref[0, pl.ds(i * _BM, _BM)] = rs

    @pl.when(j != 0)
    def _():
        row_ref[0, pl.ds(i * _BM, _BM)] += rs

    @pl.when(i == 0)
    def _():
        col_ref[0, pl.ds(j * _BM, _BM)] = cs

    @pl.when(i != 0)
    def _():
        col_ref[0, pl.ds(j * _BM, _BM)] += cs

    @pl.when(i == j)
    def _():
        diag_ref[0, pl.ds(i * _BM, _BM)] = (
            jnp.sum(z1_ref[...] * z2_ref[...], axis=1) * (1.0 / TAU))


def _contrast(z1p, z2p):
    pad = NP - N
    grid = (NP // _BM, NP // _BM)
    row, col, diag = pl.pallas_call(
        _contrast_body,
        grid=grid,
        in_specs=[
            pl.BlockSpec((_BM, Z), lambda i, j: (i, 0)),
            pl.BlockSpec((_BM, Z), lambda i, j: (j, 0)),
        ],
        out_specs=[
            pl.BlockSpec((1, NP), lambda i, j: (0, 0)),
            pl.BlockSpec((1, NP), lambda i, j: (0, 0)),
            pl.BlockSpec((1, NP), lambda i, j: (0, 0)),
        ],
        out_shape=[
            jax.ShapeDtypeStruct((1, NP), jnp.float32),
            jax.ShapeDtypeStruct((1, NP), jnp.float32),
            jax.ShapeDtypeStruct((1, NP), jnp.float32),
        ],
    )(z1p, z2p)
    return row[0, :N] - pad, col[0, :N] - pad, diag[0, :N]


def _reshape_edges(edge, ch, g):
    per_w = E // NW
    return edge.reshape(2, NW, per_w // ch, ch // g, g)


def _l2norm(x):
    return x / (jnp.linalg.norm(x, axis=1, keepdims=True) + 1e-8)


def kernel(feats_0, feats_1, feats_2, mask_feat, fc_W0, fc_b0, fc_W1, fc_b1,
           fc_W2, fc_b2, agg_W0, agg_W1, gcn_W1, gcn_b1, gcn_W2, gcn_b2,
           proj_W, proj_b, att_W, att_b, att_a, nei_edge_0, nei_edge_1,
           mask_edge_0, mask_edge_1, norm_edge_0, norm_edge_1, adj_edge,
           num_cluster):
    edges = [nei_edge_0, nei_edge_1, mask_edge_0, mask_edge_1, norm_edge_0,
             norm_edge_1, adj_edge]
    cnt_ers = [e[0].reshape(NW, _CNT_NCH, 1, _CNT_CH) for e in edges]
    cparts = _get_count7()(*cnt_ers)
    cnt = cparts[:, :, 0, :].sum(axis=1)
    recip = 1.0 / jnp.maximum(cnt, 1.0)
    rcol = [recip[v][:, None] for v in range(7)]
    r_nei0, r_nei1, r_mask0, r_mask1, r_norm0, r_norm1, r_adj = rcol

    er_nei = [_reshape_edges(e, _SS_CH, _SS_G) for e in (nei_edge_0,
                                                         nei_edge_1)]
    er5 = [_reshape_edges(e, _SS_CH, _SS_G) for e in
           (norm_edge_0, mask_edge_0, norm_edge_1, mask_edge_1, adj_edge)]
    rs5 = [r_norm0, r_mask0, r_norm1, r_mask1, r_adj]

    pad = ((0, NP - N), (0, 0))
    h_tar, h_mask, hn0, hn1 = _encode(
        jnp.pad(feats_0, pad), jnp.pad(mask_feat, pad), jnp.pad(feats_1, pad),
        jnp.pad(feats_2, pad), fc_W0, fc_b0, fc_W1, fc_b1, fc_W2, fc_b2)

    x1 = jnp.stack([hn0[:, :64], hn0[:, 64:], hn1[:, :64], hn1[:, 64:]])
    out1 = _seg_sum_batch(x1, [er_nei[0], er_nei[0], er_nei[1], er_nei[1]])

    ps = _stage2(out1, r_nei0, r_nei1, agg_W0, agg_W1, h_tar, h_mask,
                 gcn_W1, gcn_b1)

    out2 = _seg_sum_batch(ps, er5)
    qs = _stage3(out2, rs5, gcn_W2, gcn_b2)

    out3 = _seg_sum_batch(qs, er5)
    zn, z_coarse = _stage4(out3, rs5, proj_W, proj_b)

    zs = zn[:, :N]
    w = (jnp.tanh(zs @ att_W + att_b) @ att_a).mean(axis=1)
    beta = jax.nn.softmax(w)
    z_pro = _stage5(zn, beta, proj_W, proj_b)

    row_se, col_se, diag = _contrast(z_coarse, z_pro)
    l1 = (jnp.log(row_se) - diag).mean()
    l2_ = (jnp.log(col_se) - diag).mean()
    loss_info = 0.5 * (l1 + l2_)

    zp = z_pro[:N]
    assign = jnp.arange(N) % NUM_CLUSTER
    protos = _l2norm(zp.reshape(N // NUM_CLUSTER, NUM_CLUSTER, Z).mean(0))
    logits = zp @ protos.T / TAU
    pos = jnp.take_along_axis(logits, assign[:, None], axis=1)[:, 0]
    loss_proto = (jax.nn.logsumexp(logits, axis=1) - pos).mean()
    return loss_info + loss_proto
